# baseline (speedup 1.0000x reference)
import functools

import jax
import jax.numpy as jnp
from jax import lax
from jax.experimental import pallas as pl
from jax.experimental.pallas import tpu as pltpu
from jax.experimental.pallas import tpu_sc as plsc

N = 10000
E = 320000
D = 128
H = 8
C = 16
HC = H * C
NG = 64

NP = 10240
NW = 32
BE = 32
NBLK = 316
EPT = NBLK * BE
EPAD = NW * EPT
RB = 1280
GRID = NP // RB
TRASH = N
DEN0 = 10008
ND8 = (N + 7) // 8
NACC = 11264
RPT = NACC // 16

_HI = jax.lax.Precision.HIGHEST


def _proj_body(x_ref, w_ref, m_ref, h_ref, esed_ref):
    h = jax.lax.dot(x_ref[...], w_ref[...], precision=_HI)
    h_ref[...] = h
    esed_ref[...] = jax.lax.dot(h, m_ref[...], precision=_HI)


def _proj(x_pad, W, Mw):
    return pl.pallas_call(
        _proj_body,
        grid=(GRID,),
        in_specs=[
            pl.BlockSpec((RB, D), lambda i: (i, 0)),
            pl.BlockSpec((D, HC), lambda i: (0, 0)),
            pl.BlockSpec((HC, HC), lambda i: (0, 0)),
        ],
        out_specs=[
            pl.BlockSpec((RB, HC), lambda i: (i, 0)),
            pl.BlockSpec((RB, HC), lambda i: (i, 0)),
        ],
        out_shape=[
            jax.ShapeDtypeStruct((NP, HC), jnp.float32),
            jax.ShapeDtypeStruct((NP, HC), jnp.float32),
        ],
    )(x_pad, W, Mw)


def _epi_body(acc0_ref, acc1_ref, den0_ref, den1_ref, esed_ref, htab_ref,
              sw_ref, xp_ref, b_ref, out_ref, *, apply_relu):
    i = pl.program_id(0)
    num = acc0_ref[0] + acc1_ref[0]
    den16 = den0_ref[0] + den1_ref[0]
    esed = esed_ref[...][:, :16]
    htab = htab_ref[...]
    t16 = esed + jax.lax.dot(esed, sw_ref[...], precision=_HI)
    t16 = jnp.maximum(t16, 0.2 * t16)
    selfex = jnp.exp(t16)
    num = num + jax.lax.dot(selfex, xp_ref[...], precision=_HI) * htab
    den16 = den16 + selfex
    den = jax.lax.dot(den16, xp_ref[...], precision=_HI)
    out = num / (den + 1e-16) + b_ref[...]
    rows = i * RB + jax.lax.broadcasted_iota(jnp.int32, (RB, 1), 0)
    out = jnp.where(rows < N, out, 0.0)
    if apply_relu:
        out = jnp.maximum(out, 0.0)
    out_ref[...] = out


def _epilogue(acc, esed, htab, b, apply_relu):
    den = jnp.pad(acc[:, DEN0:DEN0 + ND8].reshape(2, 8 * ND8, 16),
                  ((0, 0), (0, NP - 8 * ND8), (0, 0)))
    accn = acc[:, :NP]
    p16 = jnp.arange(16)
    SW = (p16[:, None] == (p16[None, :] ^ 8)).astype(jnp.float32)
    j = jnp.arange(HC)
    XP = 0.5 * ((p16[:, None] % 8) == (j[None, :] // C)).astype(jnp.float32)
    body = functools.partial(_epi_body, apply_relu=apply_relu)
    return pl.pallas_call(
        body,
        grid=(GRID,),
        in_specs=[
            pl.BlockSpec((1, RB, HC), lambda i: (0, i, 0)),
            pl.BlockSpec((1, RB, HC), lambda i: (1, i, 0)),
            pl.BlockSpec((1, RB, 16), lambda i: (0, i, 0)),
            pl.BlockSpec((1, RB, 16), lambda i: (1, i, 0)),
            pl.BlockSpec((RB, HC), lambda i: (i, 0)),
            pl.BlockSpec((RB, HC), lambda i: (i, 0)),
            pl.BlockSpec((16, 16), lambda i: (0, 0)),
            pl.BlockSpec((16, HC), lambda i: (0, 0)),
            pl.BlockSpec((1, HC), lambda i: (0, 0)),
        ],
        out_specs=pl.BlockSpec((RB, HC), lambda i: (i, 0)),
        out_shape=jax.ShapeDtypeStruct((NP, HC), jnp.float32),
    )(accn, accn, den, den, esed, htab, SW, XP, b[None, :])


def _pool_body(x2_ref, oh_ref, fcw_ref, fcb_ref, o_ref):
    oh = oh_ref[...]
    cnt = jnp.sum(oh, axis=1, keepdims=True)
    pooled = jax.lax.dot(oh, x2_ref[...], precision=_HI)
    pooled = pooled / jnp.maximum(cnt, 1.0)
    o_ref[...] = jax.lax.dot(pooled, fcw_ref[...], precision=_HI) + fcb_ref[...]


def _pool_fc(x2, onehot, fc_W, fc_b):
    return pl.pallas_call(
        _pool_body,
        out_shape=jax.ShapeDtypeStruct((NG, 1), jnp.float32),
    )(x2, onehot, fc_W, fc_b[None, :])


_GDN = lax.GatherDimensionNumbers(
    offset_dims=(), collapsed_slice_dims=(0,), start_index_map=(0,))


def _vgather(x, idx):
    return lax.gather(x, idx[:, None], _GDN, (1,),
                      mode=lax.GatherScatterMode.PROMISE_IN_BOUNDS)

def _edge_body(htab, esed, src2, sdcat, dacat, acc_out,
               src_a, sd_a, da_a, esed_a, ybig_a,
               src_b, sd_b, da_b, esed_b, ybig_b,
               acc_sh, sem_a, sem_b, sem_ai, sem_bi):
    cid = lax.axis_index("c")
    sid = lax.axis_index("s")
    w = cid * 16 + sid

    z16 = jnp.zeros((16,), jnp.float32)

    def _zrow(r, _):
        for k in range(HC // 16):
            ybig_a[r, pl.ds(k * 16, 16)] = z16
            ybig_b[r, pl.ds(k * 16, 16)] = z16
        return 0

    lax.fori_loop(0, 2 * BE, _zrow, 0)
    base = sid * RPT
    for i in range(RPT // (2 * BE)):
        pltpu.sync_copy(ybig_a, acc_sh.at[pl.ds(base + i * 2 * BE, 2 * BE)])

    lanes = lax.iota(jnp.int32, 16)
    msk8 = lanes < 8
    swp = lanes ^ 8

    def _fire_idx(blk, sbuf, dbuf, abuf, sem):
        r = blk * BE
        pltpu.async_copy(src2.at[pl.ds(r, BE)], sbuf, sem)
        pltpu.async_copy(sdcat.at[pl.ds(2 * r, 2 * BE)], dbuf, sem)
        pltpu.async_copy(dacat.at[pl.ds(2 * r, 2 * BE)], abuf, sem)

    def _wait_idx(sbuf, dbuf, abuf, sem):
        pltpu.make_async_copy(src2.at[pl.ds(0, BE)], sbuf, sem).wait()
        pltpu.make_async_copy(sdcat.at[pl.ds(0, 2 * BE)], dbuf, sem).wait()
        pltpu.make_async_copy(dacat.at[pl.ds(0, 2 * BE)], abuf, sem).wait()

    def _fire_gather(sbuf, dbuf, ebuf, ybuf, sem):
        pltpu.async_copy(esed.at[dbuf], ebuf, sem)
        pltpu.async_copy(htab.at[sbuf], ybuf.at[pl.ds(0, BE)], sem)

    def _wait_gather(ebuf, ybuf, sem):
        pltpu.make_async_copy(esed.at[pl.ds(0, 2 * BE)], ebuf, sem).wait()
        pltpu.make_async_copy(htab.at[pl.ds(0, BE)],
                              ybuf.at[pl.ds(0, BE)], sem).wait()

    def _compute(abuf, ebuf, ybuf):
        def _chunk(ci):
            e0 = ci * 16
            qv = (abuf[pl.ds(e0, 16)] & 7) * 16
            for j in range(16):
                e = e0 + j
                vs = ebuf[e, pl.ds(0, 16)]
                vd = ebuf[BE + e, pl.ds(0, 16)]
                v = jnp.where(msk8, vs, vd)
                t = v + _vgather(v, swp)
                t = jnp.maximum(t, 0.2 * t)
                ex = jnp.exp(t)
                ybuf[BE + e, pl.ds(qv[j], 16)] = ex
                for p in range(H):
                    exp_p = _vgather(ex, jnp.full((16,), p, jnp.int32))
                    ybuf[e, pl.ds(p * C, 16)] = (
                        ybuf[e, pl.ds(p * C, 16)] * exp_p)

        for ci in range(BE // 16):
            _chunk(ci)

    def _scatter_unz(abuf, ybuf):
        pltpu.sync_copy(ybuf, acc_sh.at[abuf], add=True)

        for ci in range(BE // 16):
            e0 = ci * 16
            qv = (abuf[pl.ds(e0, 16)] & 7) * 16
            for j in range(16):
                ybuf[BE + e0 + j, pl.ds(qv[j], 16)] = z16

    plsc.subcore_barrier()

    b0 = w * NBLK
    _fire_idx(b0, src_a, sd_a, da_a, sem_ai)
    _wait_idx(src_a, sd_a, da_a, sem_ai)
    _fire_gather(src_a, sd_a, esed_a, ybig_a, sem_a)

    def _pair(g, _):
        _fire_idx(b0 + 2 * g + 1, src_b, sd_b, da_b, sem_bi)
        _wait_gather(esed_a, ybig_a, sem_a)
        _wait_idx(src_b, sd_b, da_b, sem_bi)
        _fire_gather(src_b, sd_b, esed_b, ybig_b, sem_b)
        _compute(da_a, esed_a, ybig_a)
        _scatter_unz(da_a, ybig_a)
        _fire_idx(b0 + 2 * g + 2, src_a, sd_a, da_a, sem_ai)
        _wait_gather(esed_b, ybig_b, sem_b)
        _wait_idx(src_a, sd_a, da_a, sem_ai)
        _fire_gather(src_a, sd_a, esed_a, ybig_a, sem_a)
        _compute(da_b, esed_b, ybig_b)
        _scatter_unz(da_b, ybig_b)
        return 0

    lax.fori_loop(0, NBLK // 2, _pair, 0)
    _wait_gather(esed_a, ybig_a, sem_a)

    plsc.subcore_barrier()

    pltpu.sync_copy(acc_sh.at[pl.ds(base, RPT)],
                    acc_out.at[cid, pl.ds(base, RPT)])

def _edge_aggregate(htab, esed, src2, sdcat, dacat):
    mesh = plsc.VectorSubcoreMesh(core_axis_name="c", subcore_axis_name="s")
    f = pl.kernel(
        _edge_body,
        out_type=jax.ShapeDtypeStruct((2, NACC, HC), jnp.float32),
        mesh=mesh,
        scratch_types=[
            pltpu.VMEM((BE,), jnp.int32),
            pltpu.VMEM((2 * BE,), jnp.int32),
            pltpu.VMEM((2 * BE,), jnp.int32),
            pltpu.VMEM((2 * BE, HC), jnp.float32),
            pltpu.VMEM((2 * BE, HC), jnp.float32),
            pltpu.VMEM((BE,), jnp.int32),
            pltpu.VMEM((2 * BE,), jnp.int32),
            pltpu.VMEM((2 * BE,), jnp.int32),
            pltpu.VMEM((2 * BE, HC), jnp.float32),
            pltpu.VMEM((2 * BE, HC), jnp.float32),
            pltpu.VMEM_SHARED((NACC, HC), jnp.float32),
            pltpu.SemaphoreType.DMA,
            pltpu.SemaphoreType.DMA,
            pltpu.SemaphoreType.DMA,
            pltpu.SemaphoreType.DMA,
        ],
    )
    return f(htab, esed, src2, sdcat, dacat)


def _att_matrix(a_src, a_dst):
    asf = a_src.reshape(HC)
    adf = a_dst.reshape(HC)
    j = jnp.arange(HC)
    p = jnp.arange(H)
    sel = (j[:, None] // C == p[None, :]).astype(jnp.float32)
    M = jnp.concatenate([asf[:, None] * sel, adf[:, None] * sel], axis=1)
    return jnp.pad(M, ((0, 0), (0, HC - 2 * H)))


def kernel(x, edge_index, batch, W1, a_src1, a_dst1, b1, W2, a_src2, a_dst2,
           b2, fc_W, fc_b):
    padv = jnp.full((EPAD + BE - E,), TRASH, jnp.int32)
    src3 = jnp.concatenate([edge_index[0], padv])
    dst3 = jnp.concatenate([edge_index[1], padv])
    dst8 = DEN0 + (dst3 >> 3)
    sdcat = jnp.stack([src3.reshape(-1, BE), dst3.reshape(-1, BE)],
                      axis=1).reshape(-1)
    dacat = jnp.stack([dst3.reshape(-1, BE), dst8.reshape(-1, BE)],
                      axis=1).reshape(-1)

    x_pad = jnp.pad(x, ((0, NP - N), (0, 0)))
    M1 = _att_matrix(a_src1, a_dst1)
    M2 = _att_matrix(a_src2, a_dst2)

    htab1, esed1 = _proj(x_pad, W1, M1)
    acc1 = _edge_aggregate(htab1, esed1, src3, sdcat, dacat)
    x1 = _epilogue(acc1, esed1, htab1, b1, apply_relu=True)

    htab2, esed2 = _proj(x1, W2, M2)
    acc2 = _edge_aggregate(htab2, esed2, src3, sdcat, dacat)
    x2 = _epilogue(acc2, esed2, htab2, b2, apply_relu=False)

    onehot = (batch[None, :] == jnp.arange(NG)[:, None]).astype(jnp.float32)
    out = _pool_fc(x2[:N], onehot, fc_W, fc_b)
    return out.squeeze()


# speedup vs baseline: 1.0862x; 1.0862x over previous
import functools

import jax
import jax.numpy as jnp
from jax import lax
from jax.experimental import pallas as pl
from jax.experimental.pallas import tpu as pltpu
from jax.experimental.pallas import tpu_sc as plsc

N = 10000
E = 320000
D = 128
H = 8
C = 16
HC = H * C
NG = 64

NP = 10240
NW = 32
BE = 32
NBLK = 316
EPT = NBLK * BE
EPAD = NW * EPT
RB = 1280
GRID = NP // RB
TRASH = N
DEN0 = 10008
ND8 = (N + 7) // 8
NACC = 11264
RPT = NACC // 16

_HI = jax.lax.Precision.HIGHEST


def _proj_body(x_ref, w_ref, m_ref, h_ref, esed_ref):
    h = jax.lax.dot(x_ref[...], w_ref[...], precision=_HI)
    h_ref[...] = h
    esed_ref[...] = jax.lax.dot(h, m_ref[...], precision=_HI)


def _proj(x_pad, W, Mw):
    return pl.pallas_call(
        _proj_body,
        grid=(GRID,),
        in_specs=[
            pl.BlockSpec((RB, D), lambda i: (i, 0)),
            pl.BlockSpec((D, HC), lambda i: (0, 0)),
            pl.BlockSpec((HC, HC), lambda i: (0, 0)),
        ],
        out_specs=[
            pl.BlockSpec((RB, HC), lambda i: (i, 0)),
            pl.BlockSpec((RB, HC), lambda i: (i, 0)),
        ],
        out_shape=[
            jax.ShapeDtypeStruct((NP, HC), jnp.float32),
            jax.ShapeDtypeStruct((NP, HC), jnp.float32),
        ],
    )(x_pad, W, Mw)


def _epi_body(acc0_ref, acc1_ref, den0_ref, den1_ref, esed_ref, htab_ref,
              sw_ref, xp_ref, b_ref, out_ref, *, apply_relu):
    i = pl.program_id(0)
    num = acc0_ref[0] + acc1_ref[0]
    den16 = den0_ref[0] + den1_ref[0]
    esed = esed_ref[...][:, :16]
    htab = htab_ref[...]
    t16 = esed + jax.lax.dot(esed, sw_ref[...], precision=_HI)
    t16 = jnp.maximum(t16, 0.2 * t16)
    selfex = jnp.exp(t16)
    num = num + jax.lax.dot(selfex, xp_ref[...], precision=_HI) * htab
    den16 = den16 + selfex
    den = jax.lax.dot(den16, xp_ref[...], precision=_HI)
    out = num / (den + 1e-16) + b_ref[...]
    rows = i * RB + jax.lax.broadcasted_iota(jnp.int32, (RB, 1), 0)
    out = jnp.where(rows < N, out, 0.0)
    if apply_relu:
        out = jnp.maximum(out, 0.0)
    out_ref[...] = out


def _epilogue(acc, esed, htab, b, apply_relu):
    den = jnp.pad(acc[:, DEN0:DEN0 + ND8].reshape(2, 8 * ND8, 16),
                  ((0, 0), (0, NP - 8 * ND8), (0, 0)))
    accn = acc[:, :NP]
    p16 = jnp.arange(16)
    SW = (p16[:, None] == (p16[None, :] ^ 8)).astype(jnp.float32)
    j = jnp.arange(HC)
    XP = 0.5 * ((p16[:, None] % 8) == (j[None, :] // C)).astype(jnp.float32)
    body = functools.partial(_epi_body, apply_relu=apply_relu)
    return pl.pallas_call(
        body,
        grid=(GRID,),
        in_specs=[
            pl.BlockSpec((1, RB, HC), lambda i: (0, i, 0)),
            pl.BlockSpec((1, RB, HC), lambda i: (1, i, 0)),
            pl.BlockSpec((1, RB, 16), lambda i: (0, i, 0)),
            pl.BlockSpec((1, RB, 16), lambda i: (1, i, 0)),
            pl.BlockSpec((RB, HC), lambda i: (i, 0)),
            pl.BlockSpec((RB, HC), lambda i: (i, 0)),
            pl.BlockSpec((16, 16), lambda i: (0, 0)),
            pl.BlockSpec((16, HC), lambda i: (0, 0)),
            pl.BlockSpec((1, HC), lambda i: (0, 0)),
        ],
        out_specs=pl.BlockSpec((RB, HC), lambda i: (i, 0)),
        out_shape=jax.ShapeDtypeStruct((NP, HC), jnp.float32),
    )(accn, accn, den, den, esed, htab, SW, XP, b[None, :])


def _pool_body(x2_ref, oh_ref, fcw_ref, fcb_ref, o_ref):
    oh = oh_ref[...]
    cnt = jnp.sum(oh, axis=1, keepdims=True)
    pooled = jax.lax.dot(oh, x2_ref[...], precision=_HI)
    pooled = pooled / jnp.maximum(cnt, 1.0)
    o_ref[...] = jax.lax.dot(pooled, fcw_ref[...], precision=_HI) + fcb_ref[...]


def _pool_fc(x2, onehot, fc_W, fc_b):
    return pl.pallas_call(
        _pool_body,
        out_shape=jax.ShapeDtypeStruct((NG, 1), jnp.float32),
    )(x2, onehot, fc_W, fc_b[None, :])


_GDN = lax.GatherDimensionNumbers(
    offset_dims=(), collapsed_slice_dims=(0,), start_index_map=(0,))


def _vgather(x, idx):
    return lax.gather(x, idx[:, None], _GDN, (1,),
                      mode=lax.GatherScatterMode.PROMISE_IN_BOUNDS)

def _edge_body(htab, esed, src2, sdcat, dacat, acc_out,
               src_a, sd_a, da_a, esed_a, ybig_a,
               src_b, sd_b, da_b, esed_b, ybig_b,
               acc_sh, sem_a, sem_b, sem_ai, sem_bi):
    cid = lax.axis_index("c")
    sid = lax.axis_index("s")
    w = cid * 16 + sid

    z16 = jnp.zeros((16,), jnp.float32)

    def _zrow(r, _):
        for k in range(HC // 16):
            ybig_a[r, pl.ds(k * 16, 16)] = z16
            ybig_b[r, pl.ds(k * 16, 16)] = z16
        return 0

    lax.fori_loop(0, 2 * BE, _zrow, 0)
    base = sid * RPT
    for i in range(RPT // (2 * BE)):
        pltpu.sync_copy(ybig_a, acc_sh.at[pl.ds(base + i * 2 * BE, 2 * BE)])

    lanes = lax.iota(jnp.int32, 16)
    msk8 = lanes < 8
    swp = lanes ^ 8

    def _fire_idx(blk, sbuf, dbuf, abuf, sem):
        r = blk * BE
        pltpu.async_copy(src2.at[pl.ds(r, BE)], sbuf, sem)
        pltpu.async_copy(sdcat.at[pl.ds(2 * r, 2 * BE)], dbuf, sem)
        pltpu.async_copy(dacat.at[pl.ds(2 * r, 2 * BE)], abuf, sem)

    def _wait_idx(sbuf, dbuf, abuf, sem):
        pltpu.make_async_copy(src2.at[pl.ds(0, BE)], sbuf, sem).wait()
        pltpu.make_async_copy(sdcat.at[pl.ds(0, 2 * BE)], dbuf, sem).wait()
        pltpu.make_async_copy(dacat.at[pl.ds(0, 2 * BE)], abuf, sem).wait()

    def _fire_gather(sbuf, dbuf, ebuf, ybuf, sem):
        pltpu.async_copy(esed.at[dbuf], ebuf, sem)
        pltpu.async_copy(htab.at[sbuf], ybuf.at[pl.ds(0, BE)], sem)

    def _wait_gather(ebuf, ybuf, sem):
        pltpu.make_async_copy(esed.at[pl.ds(0, 2 * BE)], ebuf, sem).wait()
        pltpu.make_async_copy(htab.at[pl.ds(0, BE)],
                              ybuf.at[pl.ds(0, BE)], sem).wait()

    def _compute(abuf, ebuf, ybuf):
        def _chunk(ci, _):
            e0 = ci * 16
            qv = (abuf[pl.ds(e0, 16)] & 7) * 16
            for j in range(16):
                e = e0 + j
                vs = ebuf[e, pl.ds(0, 16)]
                vd = ebuf[BE + e, pl.ds(0, 16)]
                v = jnp.where(msk8, vs, vd)
                t = v + _vgather(v, swp)
                t = jnp.maximum(t, 0.2 * t)
                ex = jnp.exp(t)
                ybuf[BE + e, pl.ds(qv[j], 16)] = ex
                for p in range(H):
                    exp_p = _vgather(ex, jnp.full((16,), p, jnp.int32))
                    ybuf[e, pl.ds(p * C, 16)] = (
                        ybuf[e, pl.ds(p * C, 16)] * exp_p)
            return 0

        lax.fori_loop(0, BE // 16, _chunk, 0)

    def _scatter_unz(abuf, ybuf):
        pltpu.sync_copy(ybuf, acc_sh.at[abuf], add=True)

        def _unz(ci, _):
            e0 = ci * 16
            qv = (abuf[pl.ds(e0, 16)] & 7) * 16
            for j in range(16):
                ybuf[BE + e0 + j, pl.ds(qv[j], 16)] = z16
            return 0

        lax.fori_loop(0, BE // 16, _unz, 0)

    plsc.subcore_barrier()

    b0 = w * NBLK
    _fire_idx(b0, src_a, sd_a, da_a, sem_ai)
    _wait_idx(src_a, sd_a, da_a, sem_ai)
    _fire_gather(src_a, sd_a, esed_a, ybig_a, sem_a)

    def _pair(g, _):
        _fire_idx(b0 + 2 * g + 1, src_b, sd_b, da_b, sem_bi)
        _wait_gather(esed_a, ybig_a, sem_a)
        _wait_idx(src_b, sd_b, da_b, sem_bi)
        _fire_gather(src_b, sd_b, esed_b, ybig_b, sem_b)
        _compute(da_a, esed_a, ybig_a)
        _scatter_unz(da_a, ybig_a)
        _fire_idx(b0 + 2 * g + 2, src_a, sd_a, da_a, sem_ai)
        _wait_gather(esed_b, ybig_b, sem_b)
        _wait_idx(src_a, sd_a, da_a, sem_ai)
        _fire_gather(src_a, sd_a, esed_a, ybig_a, sem_a)
        _compute(da_b, esed_b, ybig_b)
        _scatter_unz(da_b, ybig_b)
        return 0

    lax.fori_loop(0, NBLK // 2, _pair, 0)
    _wait_gather(esed_a, ybig_a, sem_a)

    plsc.subcore_barrier()

    pltpu.sync_copy(acc_sh.at[pl.ds(base, RPT)],
                    acc_out.at[cid, pl.ds(base, RPT)])

def _edge_aggregate(htab, esed, src2, sdcat, dacat):
    mesh = plsc.VectorSubcoreMesh(core_axis_name="c", subcore_axis_name="s")
    f = pl.kernel(
        _edge_body,
        out_type=jax.ShapeDtypeStruct((2, NACC, HC), jnp.float32),
        mesh=mesh,
        scratch_types=[
            pltpu.VMEM((BE,), jnp.int32),
            pltpu.VMEM((2 * BE,), jnp.int32),
            pltpu.VMEM((2 * BE,), jnp.int32),
            pltpu.VMEM((2 * BE, HC), jnp.float32),
            pltpu.VMEM((2 * BE, HC), jnp.float32),
            pltpu.VMEM((BE,), jnp.int32),
            pltpu.VMEM((2 * BE,), jnp.int32),
            pltpu.VMEM((2 * BE,), jnp.int32),
            pltpu.VMEM((2 * BE, HC), jnp.float32),
            pltpu.VMEM((2 * BE, HC), jnp.float32),
            pltpu.VMEM_SHARED((NACC, HC), jnp.float32),
            pltpu.SemaphoreType.DMA,
            pltpu.SemaphoreType.DMA,
            pltpu.SemaphoreType.DMA,
            pltpu.SemaphoreType.DMA,
        ],
    )
    return f(htab, esed, src2, sdcat, dacat)


def _att_matrix(a_src, a_dst):
    asf = a_src.reshape(HC)
    adf = a_dst.reshape(HC)
    j = jnp.arange(HC)
    p = jnp.arange(H)
    sel = (j[:, None] // C == p[None, :]).astype(jnp.float32)
    M = jnp.concatenate([asf[:, None] * sel, adf[:, None] * sel], axis=1)
    return jnp.pad(M, ((0, 0), (0, HC - 2 * H)))


def kernel(x, edge_index, batch, W1, a_src1, a_dst1, b1, W2, a_src2, a_dst2,
           b2, fc_W, fc_b):
    padv = jnp.full((EPAD + BE - E,), TRASH, jnp.int32)
    src3 = jnp.concatenate([edge_index[0], padv])
    dst3 = jnp.concatenate([edge_index[1], padv])
    dst8 = DEN0 + (dst3 >> 3)
    sdcat = jnp.stack([src3.reshape(-1, BE), dst3.reshape(-1, BE)],
                      axis=1).reshape(-1)
    dacat = jnp.stack([dst3.reshape(-1, BE), dst8.reshape(-1, BE)],
                      axis=1).reshape(-1)

    x_pad = jnp.pad(x, ((0, NP - N), (0, 0)))
    M1 = _att_matrix(a_src1, a_dst1)
    M2 = _att_matrix(a_src2, a_dst2)

    htab1, esed1 = _proj(x_pad, W1, M1)
    acc1 = _edge_aggregate(htab1, esed1, src3, sdcat, dacat)
    x1 = _epilogue(acc1, esed1, htab1, b1, apply_relu=True)

    htab2, esed2 = _proj(x1, W2, M2)
    acc2 = _edge_aggregate(htab2, esed2, src3, sdcat, dacat)
    x2 = _epilogue(acc2, esed2, htab2, b2, apply_relu=False)

    onehot = (batch[None, :] == jnp.arange(NG)[:, None]).astype(jnp.float32)
    out = _pool_fc(x2[:N], onehot, fc_W, fc_b)
    return out.squeeze()
